# baseline (speedup 1.0000x reference)
import functools

import jax
import jax.numpy as jnp
from jax import lax
from jax.experimental import pallas as pl
from jax.experimental.pallas import tpu as pltpu
from jax.experimental.pallas import tpu_sc as plsc

_D = 64
_B = 16384
_NC = 2
_NS = 16
_NW = _NC * _NS
_BPW = _B // _NW
_CHUNK = 128
_NCHUNK = _BPW // _CHUNK


@functools.partial(
    pl.kernel,
    out_type=(
        jax.ShapeDtypeStruct((_B, _D), jnp.float32),
        jax.ShapeDtypeStruct((_B, _D), jnp.float32),
    ),
    mesh=plsc.VectorSubcoreMesh(core_axis_name="c", subcore_axis_name="s"),
    compiler_params=pltpu.CompilerParams(use_tc_tiling_on_sc=False),
    scratch_types=[
        pltpu.VMEM((_NCHUNK, _CHUNK), jnp.int32),
        pltpu.VMEM((_NCHUNK, _CHUNK), jnp.int32),
        pltpu.VMEM((_BPW, _D), jnp.float32),
        pltpu.VMEM((_BPW, _D), jnp.float32),
        pltpu.SemaphoreType.DMA,
        pltpu.SemaphoreType.DMA,
    ],
)
def _dlrm_gather(user_hbm, item_hbm, uid_hbm, iid_hbm, uout_hbm, iout_hbm,
                 uidx_v, iidx_v, urows_v, irows_v, usem, isem):
    wid = lax.axis_index("s") * _NC + lax.axis_index("c")
    row0 = wid * _NCHUNK

    pltpu.sync_copy(uid_hbm.at[pl.ds(row0, _NCHUNK)], uidx_v)
    pltpu.sync_copy(iid_hbm.at[pl.ds(row0, _NCHUNK)], iidx_v)

    ucopies = []
    icopies = []
    for j in range(_NCHUNK):
        ucopies.append(pltpu.async_copy(
            user_hbm.at[uidx_v.at[j]],
            urows_v.at[pl.ds(j * _CHUNK, _CHUNK)], usem))
        icopies.append(pltpu.async_copy(
            item_hbm.at[iidx_v.at[j]],
            irows_v.at[pl.ds(j * _CHUNK, _CHUNK)], isem))
    for c in ucopies:
        c.wait()
    for c in icopies:
        c.wait()

    base = wid * _BPW
    pltpu.sync_copy(urows_v, uout_hbm.at[pl.ds(base, _BPW)])
    pltpu.sync_copy(irows_v, iout_hbm.at[pl.ds(base, _BPW)])


@jax.jit
def kernel(user_table, item_table, user_ids, item_ids):
    uid2d = user_ids.astype(jnp.int32).reshape(_B // _CHUNK, _CHUNK)
    iid2d = item_ids.astype(jnp.int32).reshape(_B // _CHUNK, _CHUNK)
    return tuple(_dlrm_gather(user_table, item_table, uid2d, iid2d))


# speedup vs baseline: 2.7641x; 2.7641x over previous
import functools

import jax
import jax.numpy as jnp
from jax import lax
from jax.experimental import pallas as pl
from jax.experimental.pallas import tpu as pltpu
from jax.experimental.pallas import tpu_sc as plsc

_D = 64
_B = 16384
_V = 1000000
_NC = 2
_NS = 16
_NW = _NC * _NS
_BPW = _B // _NW
_L = 16
_NSLOT = 8
_NROUND = _BPW // _NSLOT
_MAXBLK = _V // 128 - 1
_TAIL0 = _V - 128


def _sample_scalar(ids_v, j):
    ids16 = ids_v[pl.dslice((j >> 4) * _L, _L)]
    mask = lax.iota(jnp.int32, _L) == jnp.bitwise_and(j, _L - 1)
    return jnp.sum(jnp.where(mask, ids16, 0))


@functools.partial(
    pl.kernel,
    out_type=(
        jax.ShapeDtypeStruct((_D, _B), jnp.float32),
        jax.ShapeDtypeStruct((_D, _B), jnp.float32),
    ),
    mesh=plsc.VectorSubcoreMesh(core_axis_name="c", subcore_axis_name="s"),
    compiler_params=pltpu.CompilerParams(needs_layout_passes=False),
    scratch_types=[
        pltpu.VMEM((_BPW,), jnp.int32),
        pltpu.VMEM((_BPW,), jnp.int32),
        pltpu.VMEM((_NSLOT, 8, 8, 128), jnp.float32),
        pltpu.VMEM((_D, 128), jnp.float32),
        pltpu.VMEM((_D, _BPW), jnp.float32),
    ] + [pltpu.SemaphoreType.DMA] * _NSLOT,
)
def _dlrm_gather(user_hbm, item_hbm, utail_hbm, itail_hbm, uid_hbm, iid_hbm,
                 uout_hbm, iout_hbm, uids_v, iids_v, gbuf, tail_v, qblk,
                 s0, s1, s2, s3, s4, s5, s6, s7):
    sems = (s0, s1, s2, s3, s4, s5, s6, s7)
    wid = lax.axis_index("s") * _NC + lax.axis_index("c")
    base = wid * _BPW

    pltpu.sync_copy(uid_hbm.at[pl.ds(base, _BPW)], uids_v)
    pltpu.sync_copy(iid_hbm.at[pl.ds(base, _BPW)], iids_v)

    def do_table(tbl, tail_hbm, ids_v, out):
        pltpu.sync_copy(tail_hbm, tail_v)

        def fire(j, k):
            v = _sample_scalar(ids_v, j)
            bk = jnp.minimum(lax.shift_right_logical(v, 7), _MAXBLK)
            start = pl.multiple_of(bk * 128, 128)
            pltpu.async_copy(tbl.at[:, :, pl.ds(start, 128)],
                             gbuf.at[k], sems[k])

        def extract(j, k):
            v = _sample_scalar(ids_v, j)
            o16 = jnp.full((_L,), jnp.bitwise_and(v, 127), jnp.int32)
            ot16 = jnp.full((_L,), jnp.clip(v - _TAIL0, 0, 127), jnp.int32)
            k16 = jnp.full((_L,), k, jnp.int32)
            j16 = jnp.full((_L,), j, jnp.int32)
            in_tail = v >= _TAIL0 + 64
            for g in range(_D // _L):
                d16 = g * _L + lax.iota(jnp.int32, _L)
                a16 = lax.shift_right_logical(d16, 3)
                r16 = jnp.bitwise_and(d16, 7)
                x = plsc.load_gather(gbuf, [k16, a16, r16, o16])
                xt = plsc.load_gather(tail_v, [d16, ot16])
                x = jnp.where(in_tail, xt, x)
                plsc.store_scatter(qblk, [d16, j16], x)

        def drain(k):
            pltpu.make_async_copy(tbl.at[:, :, pl.ds(0, 128)],
                                  gbuf.at[k], sems[k]).wait()

        for k in range(_NSLOT):
            fire(k, k)

        def round_(g, _):
            for k in range(_NSLOT):
                drain(k)
                extract((g - 1) * _NSLOT + k, k)
                fire(g * _NSLOT + k, k)
            return 0
        lax.fori_loop(1, _NROUND, round_, 0)

        for k in range(_NSLOT):
            drain(k)
            extract((_NROUND - 1) * _NSLOT + k, k)

        pltpu.sync_copy(qblk, out.at[:, pl.ds(base, _BPW)])

    do_table(user_hbm, utail_hbm, uids_v, uout_hbm)
    do_table(item_hbm, itail_hbm, iids_v, iout_hbm)


@jax.jit
def kernel(user_table, item_table, user_ids, item_ids):
    ut = user_table.T.reshape(8, 8, _V)
    it = item_table.T.reshape(8, 8, _V)
    utail = user_table.T[:, _TAIL0:]
    itail = item_table.T[:, _TAIL0:]
    uid = user_ids.astype(jnp.int32)
    iid = item_ids.astype(jnp.int32)
    uT, iT = _dlrm_gather(ut, it, utail, itail, uid, iid)
    return (uT.T, iT.T)


# speedup vs baseline: 2.7906x; 1.0096x over previous
import functools

import jax
import jax.numpy as jnp
from jax import lax
from jax.experimental import pallas as pl
from jax.experimental.pallas import tpu as pltpu
from jax.experimental.pallas import tpu_sc as plsc

_D = 64
_B = 16384
_V = 1000000
_NC = 2
_NS = 16
_NW = _NC * _NS
_BPW = _B // _NW
_L = 16
_NSLOT = 8
_NROUND = _BPW // _NSLOT
_MAXBLK = _V // 128 - 1
_TAIL0 = _V - 128


def _scal(ids_v, j):
    ids16 = ids_v[pl.dslice((j >> 4) * _L, _L)]
    mask = lax.iota(jnp.int32, _L) == jnp.bitwise_and(j, _L - 1)
    return jnp.sum(jnp.where(mask, ids16, 0))


@functools.partial(
    pl.kernel,
    out_type=(
        jax.ShapeDtypeStruct((_D, _B), jnp.float32),
        jax.ShapeDtypeStruct((_D, _B), jnp.float32),
    ),
    mesh=plsc.VectorSubcoreMesh(core_axis_name="c", subcore_axis_name="s"),
    compiler_params=pltpu.CompilerParams(needs_layout_passes=False),
    scratch_types=[
        pltpu.VMEM((_BPW + _L,), jnp.int32),
        pltpu.VMEM((_BPW + _L,), jnp.int32),
        pltpu.VMEM((_NSLOT, 8, 8, 128), jnp.float32),
        pltpu.VMEM((_D, 128), jnp.float32),
        pltpu.VMEM((_D, _BPW), jnp.float32),
    ] + [pltpu.SemaphoreType.DMA] * _NSLOT,
)
def _dlrm_gather(user_hbm, item_hbm, utail_hbm, itail_hbm, uid_hbm, iid_hbm,
                 uout_hbm, iout_hbm, uids_v, iids_v, gbuf, tail_v, qblk,
                 s0, s1, s2, s3, s4, s5, s6, s7):
    sems = (s0, s1, s2, s3, s4, s5, s6, s7)
    wid = lax.axis_index("s") * _NC + lax.axis_index("c")
    base = wid * _BPW

    pltpu.sync_copy(uid_hbm.at[pl.ds(base, _BPW)], uids_v.at[pl.ds(0, _BPW)])
    pltpu.sync_copy(iid_hbm.at[pl.ds(base, _BPW)], iids_v.at[pl.ds(0, _BPW)])

    iota = lax.iota(jnp.int32, _L)
    d16s = [g * _L + iota for g in range(_D // _L)]
    a16s = [lax.shift_right_logical(d, 3) for d in d16s]
    r16s = [jnp.bitwise_and(d, 7) for d in d16s]
    k16s = [jnp.full((_L,), k, jnp.int32) for k in range(_NSLOT)]

    def do_table(tbl, tail_hbm, ids_v, out):
        pltpu.sync_copy(tail_hbm, tail_v)

        def fire(j, k):
            v = _scal(ids_v, j)
            bk = jnp.minimum(lax.shift_right_logical(v, 7), _MAXBLK)
            start = pl.multiple_of(bk * 128, 128)
            pltpu.async_copy(tbl.at[:, :, pl.ds(start, 128)],
                             gbuf.at[k], sems[k])

        def extract(j, k):
            v = _scal(ids_v, j)
            o16 = jnp.full((_L,), jnp.bitwise_and(v, 127), jnp.int32)
            j16 = jnp.full((_L,), j, jnp.int32)
            for g in range(_D // _L):
                x = plsc.load_gather(gbuf, [k16s[k], a16s[g], r16s[g], o16])
                plsc.store_scatter(qblk, [d16s[g], j16], x)

            @pl.when(v >= _TAIL0 + 64)
            def _patch_tail():
                ot16 = jnp.full((_L,), v - _TAIL0, jnp.int32)
                for g in range(_D // _L):
                    xt = plsc.load_gather(tail_v, [d16s[g], ot16])
                    plsc.store_scatter(qblk, [d16s[g], j16], xt)

        def drain(k):
            pltpu.make_async_copy(tbl.at[:, :, pl.ds(0, 128)],
                                  gbuf.at[k], sems[k]).wait()

        for k in range(_NSLOT):
            fire(k, k)

        def round_(g, _):
            for k in range(_NSLOT):
                drain(k)
                extract((g - 1) * _NSLOT + k, k)
                fire(g * _NSLOT + k, k)
            return 0
        lax.fori_loop(1, _NROUND, round_, 0)

        for k in range(_NSLOT):
            drain(k)
            extract((_NROUND - 1) * _NSLOT + k, k)

        pltpu.sync_copy(qblk, out.at[:, pl.ds(base, _BPW)])

    do_table(user_hbm, utail_hbm, uids_v, uout_hbm)
    do_table(item_hbm, itail_hbm, iids_v, iout_hbm)


@jax.jit
def kernel(user_table, item_table, user_ids, item_ids):
    ut = user_table.T.reshape(8, 8, _V)
    it = item_table.T.reshape(8, 8, _V)
    utail = user_table.T[:, _TAIL0:]
    itail = item_table.T[:, _TAIL0:]
    uid = user_ids.astype(jnp.int32)
    iid = item_ids.astype(jnp.int32)
    uT, iT = _dlrm_gather(ut, it, utail, itail, uid, iid)
    return (uT.T, iT.T)


# speedup vs baseline: 2.7948x; 1.0015x over previous
import functools

import jax
import jax.numpy as jnp
from jax import lax
from jax.experimental import pallas as pl
from jax.experimental.pallas import tpu as pltpu
from jax.experimental.pallas import tpu_sc as plsc

_D = 64
_B = 16384
_V = 1000000
_NC = 2
_NS = 16
_NW = _NC * _NS
_BPW = _B // _NW
_L = 16
_NSLOT = 8
_NROUND = _BPW // _NSLOT
_MAXBLK = _V // 128 - 1
_TAIL0 = _V - 128


def _scal(ids_v, j):
    ids16 = ids_v[pl.dslice((j >> 4) * _L, _L)]
    mask = lax.iota(jnp.int32, _L) == jnp.bitwise_and(j, _L - 1)
    return jnp.sum(jnp.where(mask, ids16, 0))


@functools.partial(
    pl.kernel,
    out_type=(
        jax.ShapeDtypeStruct((_D, _B), jnp.float32),
        jax.ShapeDtypeStruct((_D, _B), jnp.float32),
    ),
    mesh=plsc.VectorSubcoreMesh(core_axis_name="c", subcore_axis_name="s"),
    compiler_params=pltpu.CompilerParams(needs_layout_passes=False),
    scratch_types=[
        pltpu.VMEM((_BPW + _L,), jnp.int32),
        pltpu.VMEM((_BPW + _L,), jnp.int32),
        pltpu.VMEM((_NSLOT, 8, 8, 128), jnp.float32),
        pltpu.VMEM((_D, 128), jnp.float32),
        pltpu.VMEM((_D, _BPW), jnp.float32),
    ] + [pltpu.SemaphoreType.DMA] * _NSLOT,
)
def _dlrm_gather(user_hbm, item_hbm, utail_hbm, itail_hbm, uid_hbm, iid_hbm,
                 uout_hbm, iout_hbm, uids_v, iids_v, gbuf, tail_v, qblk,
                 s0, s1, s2, s3, s4, s5, s6, s7):
    sems = (s0, s1, s2, s3, s4, s5, s6, s7)
    wid = lax.axis_index("s") * _NC + lax.axis_index("c")
    base = wid * _BPW

    pltpu.sync_copy(uid_hbm.at[pl.ds(base, _BPW)], uids_v.at[pl.ds(0, _BPW)])
    pltpu.sync_copy(iid_hbm.at[pl.ds(base, _BPW)], iids_v.at[pl.ds(0, _BPW)])

    iota = lax.iota(jnp.int32, _L)
    d16s = [g * _L + iota for g in range(_D // _L)]
    a16s = [lax.shift_right_logical(d, 3) for d in d16s]
    r16s = [jnp.bitwise_and(d, 7) for d in d16s]

    def do_table(tbl, tail_hbm, ids_v, out):
        pltpu.sync_copy(tail_hbm, tail_v)

        def fire(j, k):
            v = _scal(ids_v, j)
            bk = jnp.minimum(lax.shift_right_logical(v, 7), _MAXBLK)
            start = pl.multiple_of(bk * 128, 128)
            pltpu.async_copy(tbl.at[:, :, pl.ds(start, 128)],
                             gbuf.at[k], sems[k])

        def extract(j, k):
            v = _scal(ids_v, j)
            o16 = jnp.full((_L,), jnp.bitwise_and(v, 127), jnp.int32)
            j16 = jnp.full((_L,), j, jnp.int32)
            for g in range(_D // _L):
                x = plsc.load_gather(gbuf.at[k], [a16s[g], r16s[g], o16])
                plsc.store_scatter(qblk, [d16s[g], j16], x)

            @pl.when(v >= _TAIL0 + 64)
            def _patch_tail():
                ot16 = jnp.full((_L,), v - _TAIL0, jnp.int32)
                for g in range(_D // _L):
                    xt = plsc.load_gather(tail_v, [d16s[g], ot16])
                    plsc.store_scatter(qblk, [d16s[g], j16], xt)

        def drain(k):
            pltpu.make_async_copy(tbl.at[:, :, pl.ds(0, 128)],
                                  gbuf.at[k], sems[k]).wait()

        for k in range(_NSLOT):
            fire(k, k)

        def round_(g, _):
            for k in range(_NSLOT):
                drain(k)
                extract((g - 1) * _NSLOT + k, k)
                fire(g * _NSLOT + k, k)
            return 0
        lax.fori_loop(1, _NROUND, round_, 0)

        for k in range(_NSLOT):
            drain(k)
            extract((_NROUND - 1) * _NSLOT + k, k)

        pltpu.sync_copy(qblk, out.at[:, pl.ds(base, _BPW)])

    do_table(user_hbm, utail_hbm, uids_v, uout_hbm)
    do_table(item_hbm, itail_hbm, iids_v, iout_hbm)


@jax.jit
def kernel(user_table, item_table, user_ids, item_ids):
    ut = user_table.T.reshape(8, 8, _V)
    it = item_table.T.reshape(8, 8, _V)
    utail = user_table.T[:, _TAIL0:]
    itail = item_table.T[:, _TAIL0:]
    uid = user_ids.astype(jnp.int32)
    iid = item_ids.astype(jnp.int32)
    uT, iT = _dlrm_gather(ut, it, utail, itail, uid, iid)
    return (uT.T, iT.T)
